# baseline (speedup 1.0000x reference)
import functools

import jax
import jax.numpy as jnp
from jax import lax
from jax.experimental import pallas as pl
from jax.experimental.pallas import tpu as pltpu
from jax.experimental.pallas import tpu_sc as plsc

VOCAB = 89
BATCH = 4096
HIST = 50
EMBED = 64
TOTAL = BATCH * HIST
PACKED = TOTAL // 2
NUM_WORKERS = 32
PER_WORKER = TOTAL // NUM_WORKERS
CHUNK = 128
PCHUNK = CHUNK // 2
NCHUNKS = PER_WORKER // CHUNK
NBUF = 6
LOOKAHEAD = 3

_mesh = plsc.VectorSubcoreMesh(core_axis_name="c", subcore_axis_name="s")


@functools.partial(
    pl.kernel,
    out_type=jax.ShapeDtypeStruct((PACKED, 128), jnp.float32),
    mesh=_mesh,
    scratch_types=[
        pltpu.VMEM((NCHUNKS, CHUNK), jnp.int32),
        pltpu.VMEM((NBUF, CHUNK, EMBED), jnp.float32),
        pltpu.VMEM((VOCAB, EMBED), jnp.float32),
        pltpu.VMEM_SHARED((VOCAB, EMBED), jnp.float32),
        pltpu.SemaphoreType.DMA,
        pltpu.SemaphoreType.DMA,
    ],
    compiler_params=pltpu.CompilerParams(use_tc_tiling_on_sc=False),
)
def _emb_lookup(idx_hbm, table_hbm, out_hbm, idx_v, rows_v, tab_v, tab_sh, gsem, wsem):
    wid = lax.axis_index("s") * 2 + lax.axis_index("c")
    pbase = wid * (PER_WORKER // 2)

    @pl.when(lax.axis_index("s") == 0)
    def _():
        pltpu.sync_copy(table_hbm, tab_v)
        pltpu.sync_copy(tab_v, tab_sh)

    plsc.subcore_barrier()

    pltpu.sync_copy(idx_hbm.at[wid], idx_v)

    def gather(j, buf):
        pltpu.async_copy(tab_sh.at[idx_v.at[j]], rows_v.at[buf], gsem)

    def write(j, buf):
        pltpu.async_copy(
            rows_v.at[buf, pl.ds(0, PCHUNK)],
            out_hbm.at[pl.ds(pbase + j * PCHUNK, PCHUNK), pl.ds(0, EMBED)],
            wsem,
        )
        pltpu.async_copy(
            rows_v.at[buf, pl.ds(PCHUNK, PCHUNK)],
            out_hbm.at[pl.ds(pbase + j * PCHUNK, PCHUNK), pl.ds(EMBED, EMBED)],
            wsem,
        )

    def wait_gather():
        pltpu.make_async_copy(
            out_hbm.at[pl.ds(pbase, CHUNK), pl.ds(0, EMBED)],
            rows_v.at[0],
            gsem,
        ).wait()

    def wait_write():
        for half in range(2):
            pltpu.make_async_copy(
                rows_v.at[0, pl.ds(0, PCHUNK)],
                out_hbm.at[pl.ds(pbase, PCHUNK), pl.ds(half * EMBED, EMBED)],
                wsem,
            ).wait()

    for b in range(LOOKAHEAD):
        gather(b, b)

    def body(j, carry):
        wait_gather()
        nj = j + LOOKAHEAD

        @pl.when(nj < NCHUNKS)
        def _():
            @pl.when(nj >= NBUF)
            def _():
                wait_write()

            gather(nj, lax.rem(nj, NBUF))

        write(j, lax.rem(j, NBUF))
        return carry

    lax.fori_loop(0, NCHUNKS, body, 0)

    for _ in range(NBUF):
        wait_write()


BB = 128


def _repack_body(i_ref, o_ref):
    x = i_ref[...]
    left = x[:, :EMBED][:, None, :]
    right = x[:, EMBED:][:, None, :]
    y = jnp.concatenate([left, right], axis=1)
    o_ref[...] = y.reshape(BB, HIST, EMBED)


_repack = pl.pallas_call(
    _repack_body,
    grid=(BATCH // BB,),
    in_specs=[pl.BlockSpec((BB * HIST // 2, 128), lambda i: (i, 0))],
    out_specs=pl.BlockSpec((BB, HIST, EMBED), lambda i: (i, 0, 0)),
    out_shape=jax.ShapeDtypeStruct((BATCH, HIST, EMBED), jnp.float32),
)


def kernel(x, table):
    idx = (
        x.reshape(NUM_WORKERS, NCHUNKS, PCHUNK, 2)
        .swapaxes(-1, -2)
        .reshape(NUM_WORKERS, NCHUNKS, CHUNK)
        .astype(jnp.int32)
    )
    packed = _emb_lookup(idx, table)
    return _repack(packed)


# speedup vs baseline: 1.6459x; 1.6459x over previous
import functools

import jax
import jax.numpy as jnp
from jax import lax
from jax.experimental import pallas as pl
from jax.experimental.pallas import tpu as pltpu
from jax.experimental.pallas import tpu_sc as plsc

VOCAB = 89
BATCH = 4096
HIST = 50
EMBED = 64
TOTAL = BATCH * HIST
NUM_WORKERS = 32
PER_WORKER = TOTAL // NUM_WORKERS
CHUNK = 128
NCHUNKS = PER_WORKER // CHUNK
NBUF = 6
LOOKAHEAD = 3

_mesh = plsc.VectorSubcoreMesh(core_axis_name="c", subcore_axis_name="s")


@functools.partial(
    pl.kernel,
    out_type=jax.ShapeDtypeStruct((TOTAL, EMBED), jnp.float32),
    mesh=_mesh,
    scratch_types=[
        pltpu.VMEM((NCHUNKS, CHUNK), jnp.int32),
        pltpu.VMEM((NBUF, CHUNK, EMBED), jnp.float32),
        pltpu.VMEM((VOCAB, EMBED), jnp.float32),
        pltpu.VMEM_SHARED((VOCAB, EMBED), jnp.float32),
        pltpu.SemaphoreType.DMA,
        pltpu.SemaphoreType.DMA,
    ],
    compiler_params=pltpu.CompilerParams(use_tc_tiling_on_sc=False),
)
def _emb_lookup(idx_hbm, table_hbm, out_hbm, idx_v, rows_v, tab_v, tab_sh, gsem, wsem):
    wid = lax.axis_index("s") * 2 + lax.axis_index("c")
    base = wid * PER_WORKER

    @pl.when(lax.axis_index("s") == 0)
    def _():
        pltpu.sync_copy(table_hbm, tab_v)
        pltpu.sync_copy(tab_v, tab_sh)

    plsc.subcore_barrier()

    pltpu.sync_copy(idx_hbm.at[wid], idx_v)

    def gather(j, buf):
        pltpu.async_copy(tab_sh.at[idx_v.at[j]], rows_v.at[buf], gsem)

    def write(j, buf):
        pltpu.async_copy(
            rows_v.at[buf], out_hbm.at[pl.ds(base + j * CHUNK, CHUNK)], wsem
        )

    def wait_gather():
        pltpu.make_async_copy(
            out_hbm.at[pl.ds(base, CHUNK)], rows_v.at[0], gsem
        ).wait()

    def wait_write():
        pltpu.make_async_copy(
            rows_v.at[0], out_hbm.at[pl.ds(base, CHUNK)], wsem
        ).wait()

    for b in range(LOOKAHEAD):
        gather(b, b)

    def body(j, carry):
        wait_gather()
        nj = j + LOOKAHEAD

        @pl.when(nj < NCHUNKS)
        def _():
            @pl.when(nj >= NBUF)
            def _():
                wait_write()

            gather(nj, lax.rem(nj, NBUF))

        write(j, lax.rem(j, NBUF))
        return carry

    lax.fori_loop(0, NCHUNKS, body, 0)

    for _ in range(NBUF):
        wait_write()


def kernel(x, table):
    idx = x.reshape(NUM_WORKERS, NCHUNKS, CHUNK).astype(jnp.int32)
    out = _emb_lookup(idx, table)
    return out.reshape(BATCH, HIST, EMBED)
